# baseline (speedup 1.0000x reference)
import jax
import jax.numpy as jnp
from jax.experimental import pallas as pl
from jax.experimental.pallas import tpu as pltpu

FEATURE_DIMS = (20000, 5000, 3000)
FDPAD = (20000, 5120, 3072)
UNMASKED = (500, 200, 100)
GRID = 20000
MASK = 512
H = 64
BM = 128
NB = MASK // BM
BK = 1000
_C0 = FEATURE_DIMS[0] // BK
_C1 = _C0 + FEATURE_DIMS[1] // BK
_C2 = _C1 + FEATURE_DIMS[2] // BK
_DN = (((1,), (0,)), ((), ()))
_DT = (((1,), (1,)), ((), ()))


def _stage_a_body(x_hbm, w0_ref, w1_ref, w2_ref, o0_ref, o1_ref, o2_ref,
                  xb0, xb1, xb2, sems):
    m = pl.program_id(0)
    bufs = (xb0, xb1, xb2)

    def copies(mm, sl):
        out = [pltpu.make_async_copy(
            x_hbm.at[0, pl.ds(mm * BM, BM)], bufs[0].at[sl], sems.at[sl, 0])]
        for o in (1, 2):
            out.append(pltpu.make_async_copy(
                x_hbm.at[o, pl.ds(mm * BM, BM), pl.ds(0, FDPAD[o])],
                bufs[o].at[sl], sems.at[sl, o]))
        return out

    @pl.when(m == 0)
    def _():
        for c in copies(0, 0):
            c.start()

    @pl.when(m + 1 < NB)
    def _():
        for c in copies(m + 1, (m + 1) % 2):
            c.start()

    slot = m % 2
    for c in copies(m, slot):
        c.wait()
    for buf, w_ref, out_ref in zip(
            bufs, (w0_ref, w1_ref, w2_ref), (o0_ref, o1_ref, o2_ref)):
        out_ref[...] = jax.lax.dot_general(
            buf[slot], w_ref[...], _DN, preferred_element_type=jnp.float32)


def _stage_a(x2d, w0, w1p, w2p):
    hbm = pl.BlockSpec(memory_space=pltpu.MemorySpace.HBM)
    wspec = lambda fd: pl.BlockSpec((fd, H), lambda m: (0, 0))
    ospec = pl.BlockSpec((BM, H), lambda m: (m, 0))
    oshape = jax.ShapeDtypeStruct((MASK, H), jnp.float32)
    return pl.pallas_call(
        _stage_a_body,
        grid=(NB,),
        in_specs=[hbm, wspec(FDPAD[0]), wspec(FDPAD[1]), wspec(FDPAD[2])],
        out_specs=[ospec, ospec, ospec],
        out_shape=[oshape, oshape, oshape],
        scratch_shapes=[
            pltpu.VMEM((2, BM, FDPAD[0]), jnp.float32),
            pltpu.VMEM((2, BM, FDPAD[1]), jnp.float32),
            pltpu.VMEM((2, BM, FDPAD[2]), jnp.float32),
            pltpu.SemaphoreType.DMA((2, 3)),
        ],
    )(x2d, w0, w1p, w2p)


def _stage_b_body(pre0_ref, pre1_ref, pre2_ref, b0_ref, b1_ref, b2_ref,
                  wo0_ref, bo0_ref, wo1_ref, bo1_ref, wo2t_ref,
                  scores_ref, gp0_ref, gp1_ref, gp2_ref):
    pres = (pre0_ref[...] + b0_ref[...], pre1_ref[...] + b1_ref[...],
            pre2_ref[...] + b2_ref[...])
    ms = tuple(p > 0.0 for p in pres)
    hcat = jnp.concatenate([jnp.maximum(p, 0.0) for p in pres], axis=1)

    a0 = jax.lax.dot_general(hcat, wo0_ref[...], _DN,
                             preferred_element_type=jnp.float32) + bo0_ref[...]
    m0 = a0 > 0.0
    h0 = jnp.maximum(a0, 0.0)

    a1 = jax.lax.dot_general(h0, wo1_ref[...], _DN,
                             preferred_element_type=jnp.float32) + bo1_ref[...]
    m1 = a1 > 0.0
    h1 = jnp.maximum(a1, 0.0)

    wo2t = wo2t_ref[...]
    scores_ref[...] = jax.lax.dot_general(
        h1, wo2t, _DT, preferred_element_type=jnp.float32)

    g1 = jnp.where(m1, wo2t, 0.0)
    g0 = jax.lax.dot_general(g1, wo1_ref[...], _DT,
                             preferred_element_type=jnp.float32)
    g0 = jnp.where(m0, g0, 0.0)
    for i, gp_ref in enumerate((gp0_ref, gp1_ref, gp2_ref)):
        gc_i = jax.lax.dot_general(
            g0, wo0_ref[i * H:(i + 1) * H, :], _DT,
            preferred_element_type=jnp.float32)
        gp_ref[...] = jnp.where(ms[i], gc_i, 0.0)


def _stage_b(pre0, pre1, pre2, b0, b1, b2, wo0, bo0, wo1, bo1, wo2t):
    full = lambda s: pl.BlockSpec(s, lambda: tuple(0 for _ in s))
    gshape = jax.ShapeDtypeStruct((MASK, H), jnp.float32)
    return pl.pallas_call(
        _stage_b_body,
        in_specs=[full((MASK, H)), full((MASK, H)), full((MASK, H)),
                  full((1, H)), full((1, H)), full((1, H)),
                  full((3 * H, 128)), full((1, 128)),
                  full((128, 32)), full((1, 32)), full((1, 32))],
        out_specs=[full((MASK, 1)), full((MASK, H)), full((MASK, H)),
                   full((MASK, H))],
        out_shape=[jax.ShapeDtypeStruct((MASK, 1), jnp.float32),
                   gshape, gshape, gshape],
    )(pre0, pre1, pre2, b0, b1, b2, wo0, bo0, wo1, bo1, wo2t)


def _stage_c_body(w0_ref, w1_ref, w2_ref, gp0_ref, gp1_ref, gp2_ref, imp_ref):
    g = pl.program_id(0)
    o = (g >= _C0).astype(jnp.int32) + (g >= _C1).astype(jnp.int32)
    for i, (w_ref, gp_ref) in enumerate(
            zip((w0_ref, w1_ref, w2_ref), (gp0_ref, gp1_ref, gp2_ref))):
        @pl.when(o == i)
        def _():
            gt = jax.lax.dot_general(w_ref[...], gp_ref[...], _DT,
                                     preferred_element_type=jnp.float32)
            ones = jnp.full((MASK, 1), -1.0 / MASK, jnp.float32)
            imp_ref[...] = jax.lax.dot_general(
                gt, ones, _DN, preferred_element_type=jnp.float32,
                precision=jax.lax.Precision.HIGHEST)


def _stage_c(w0, w1, w2, gp0, gp1, gp2):
    full = lambda s: pl.BlockSpec(s, lambda: tuple(0 for _ in s[:2]))
    wspec = [
        pl.BlockSpec((BK, H), lambda g: (jnp.minimum(g, _C0 - 1), 0)),
        pl.BlockSpec((BK, H),
                     lambda g: (jnp.clip(g - _C0, 0, _C1 - _C0 - 1), 0)),
        pl.BlockSpec((BK, H),
                     lambda g: (jnp.clip(g - _C1, 0, _C2 - _C1 - 1), 0)),
    ]
    gspec = pl.BlockSpec((MASK, H), lambda g: (0, 0))
    return pl.pallas_call(
        _stage_c_body,
        grid=(_C2,),
        in_specs=wspec + [gspec, gspec, gspec],
        out_specs=pl.BlockSpec((BK, 1), lambda g: (g, 0)),
        out_shape=jax.ShapeDtypeStruct((_C2 * BK, 1), jnp.float32),
    )(w0, w1, w2, gp0, gp1, gp2)


def _sortable_key(f):
    b = jax.lax.bitcast_convert_type(f, jnp.uint32)
    neg = b >= jnp.uint32(0x80000000)
    return jnp.where(neg, ~b, b | jnp.uint32(0x80000000))


def _kth_key(key, k):
    kk = jnp.int32(k)

    def bit_step(i, t):
        cand = t | (jnp.uint32(1) << (jnp.uint32(31) - i.astype(jnp.uint32)))
        cnt = jnp.sum((key >= cand).astype(jnp.int32))
        return jnp.where(cnt >= kk, cand, t)

    return jax.lax.fori_loop(0, 32, bit_step, jnp.uint32(0))


def _stage_d_body(imp0_ref, imp1_ref, imp2_ref, mask_ref):
    rows = []
    for ref, fd, k in zip((imp0_ref, imp1_ref, imp2_ref),
                          FEATURE_DIMS, UNMASKED):
        key = _sortable_key(ref[...])
        row = (key >= _kth_key(key, k)).astype(jnp.float32)
        if fd < GRID:
            row = jnp.concatenate(
                [row, jnp.zeros((1, GRID - fd), jnp.float32)], axis=1)
        rows.append(row)
    mask_ref[...] = jnp.concatenate(rows, axis=0)


def _stage_d(imp0, imp1, imp2):
    full = lambda s: pl.BlockSpec(s, lambda: (0, 0))
    return pl.pallas_call(
        _stage_d_body,
        in_specs=[full((1, FEATURE_DIMS[0])), full((1, FEATURE_DIMS[1])),
                  full((1, FEATURE_DIMS[2]))],
        out_specs=full((3, GRID)),
        out_shape=jax.ShapeDtypeStruct((3, GRID), jnp.float32),
    )(imp0, imp1, imp2)


def kernel(x, W0, b0, W1, b1, W2, b2, Wo0, bo0, Wo1, bo1, Wo2, bo2):
    w1p = jnp.pad(W1, ((0, FDPAD[1] - FEATURE_DIMS[1]), (0, 0)))
    w2p = jnp.pad(W2, ((0, FDPAD[2] - FEATURE_DIMS[2]), (0, 0)))
    pre0, pre1, pre2 = _stage_a(x, W0, w1p, w2p)

    scores, gp0, gp1, gp2 = _stage_b(
        pre0, pre1, pre2, b0[None, :], b1[None, :], b2[None, :],
        Wo0, bo0[None, :], Wo1, bo1[None, :], Wo2.T)

    imp = _stage_c(W0, W1, W2, gp0, gp1, gp2)
    imp_row = imp.reshape(1, _C2 * BK)
    imp0 = jax.lax.slice(imp_row, (0, 0), (1, FEATURE_DIMS[0]))
    imp1 = jax.lax.slice(imp_row, (0, _C0 * BK), (1, _C1 * BK))
    imp2 = jax.lax.slice(imp_row, (0, _C1 * BK), (1, _C2 * BK))
    mask_opt = _stage_d(imp0, imp1, imp2)
    return scores + bo2[None, :], mask_opt


# speedup vs baseline: 2.1793x; 2.1793x over previous
import jax
import jax.numpy as jnp
from jax.experimental import pallas as pl

FEATURE_DIMS = (20000, 5000, 3000)
UNMASKED = (500, 200, 100)
GRID = 20000
MASK = 512
H = 64
BK = 1000
_C0 = FEATURE_DIMS[0] // BK
_C1 = _C0 + FEATURE_DIMS[1] // BK
_C2 = _C1 + FEATURE_DIMS[2] // BK
_DN = (((1,), (0,)), ((), ()))
_DT = (((1,), (1,)), ((), ()))
_D00 = (((0,), (0,)), ((), ()))


def _omic(g):
    return (g >= _C0).astype(jnp.int32) + (g >= _C1).astype(jnp.int32)


def _kloc(g):
    return g - jnp.where(g >= _C1, _C1, jnp.where(g >= _C0, _C0, 0))


_WSPECS = [
    pl.BlockSpec((BK, H), lambda g: (jnp.clip(g, 0, _C0 - 1), 0)),
    pl.BlockSpec((BK, H), lambda g: (jnp.clip(g - _C0, 0, _C1 - _C0 - 1), 0)),
    pl.BlockSpec((BK, H), lambda g: (jnp.clip(g - _C1, 0, _C2 - _C1 - 1), 0)),
]


def _stage_a_body(xt_ref, w0_ref, w1_ref, w2_ref, o0_ref, o1_ref, o2_ref):
    g = pl.program_id(0)
    o = _omic(g)
    xb = xt_ref[...][0]
    for i, (w_ref, out_ref) in enumerate(
            zip((w0_ref, w1_ref, w2_ref), (o0_ref, o1_ref, o2_ref))):
        @pl.when(o == i)
        def _():
            acc = jax.lax.dot_general(
                xb, w_ref[...], _D00, preferred_element_type=jnp.float32)

            @pl.when(_kloc(g) == 0)
            def _():
                out_ref[...] = acc

            @pl.when(_kloc(g) != 0)
            def _():
                out_ref[...] += acc


def _stage_a(xt, w0, w1, w2):
    ospec = pl.BlockSpec((MASK, H), lambda g: (0, 0))
    oshape = jax.ShapeDtypeStruct((MASK, H), jnp.float32)
    return pl.pallas_call(
        _stage_a_body,
        grid=(_C2,),
        in_specs=[pl.BlockSpec((1, BK, MASK),
                               lambda g: (_omic(g), _kloc(g), 0))] + _WSPECS,
        out_specs=[ospec, ospec, ospec],
        out_shape=[oshape, oshape, oshape],
    )(xt, w0, w1, w2)


def _stage_b_body(pre0_ref, pre1_ref, pre2_ref, b0_ref, b1_ref, b2_ref,
                  wo0_ref, bo0_ref, wo1_ref, bo1_ref, wo2t_ref,
                  scores_ref, gp0_ref, gp1_ref, gp2_ref):
    pres = (pre0_ref[...] + b0_ref[...], pre1_ref[...] + b1_ref[...],
            pre2_ref[...] + b2_ref[...])
    ms = tuple(p > 0.0 for p in pres)
    hcat = jnp.concatenate([jnp.maximum(p, 0.0) for p in pres], axis=1)

    a0 = jax.lax.dot_general(hcat, wo0_ref[...], _DN,
                             preferred_element_type=jnp.float32) + bo0_ref[...]
    m0 = a0 > 0.0
    h0 = jnp.maximum(a0, 0.0)

    a1 = jax.lax.dot_general(h0, wo1_ref[...], _DN,
                             preferred_element_type=jnp.float32) + bo1_ref[...]
    m1 = a1 > 0.0
    h1 = jnp.maximum(a1, 0.0)

    wo2t = wo2t_ref[...]
    scores_ref[...] = jax.lax.dot_general(
        h1, wo2t, _DT, preferred_element_type=jnp.float32)

    g1 = jnp.where(m1, wo2t, 0.0)
    g0 = jax.lax.dot_general(g1, wo1_ref[...], _DT,
                             preferred_element_type=jnp.float32)
    g0 = jnp.where(m0, g0, 0.0)
    for i, gp_ref in enumerate((gp0_ref, gp1_ref, gp2_ref)):
        gc_i = jax.lax.dot_general(
            g0, wo0_ref[i * H:(i + 1) * H, :], _DT,
            preferred_element_type=jnp.float32)
        gp_ref[...] = jnp.where(ms[i], gc_i, 0.0)


def _stage_b(pre0, pre1, pre2, b0, b1, b2, wo0, bo0, wo1, bo1, wo2t):
    full = lambda s: pl.BlockSpec(s, lambda: tuple(0 for _ in s))
    gshape = jax.ShapeDtypeStruct((MASK, H), jnp.float32)
    return pl.pallas_call(
        _stage_b_body,
        in_specs=[full((MASK, H)), full((MASK, H)), full((MASK, H)),
                  full((1, H)), full((1, H)), full((1, H)),
                  full((3 * H, 128)), full((1, 128)),
                  full((128, 32)), full((1, 32)), full((1, 32))],
        out_specs=[full((MASK, 1)), full((MASK, H)), full((MASK, H)),
                   full((MASK, H))],
        out_shape=[jax.ShapeDtypeStruct((MASK, 1), jnp.float32),
                   gshape, gshape, gshape],
    )(pre0, pre1, pre2, b0, b1, b2, wo0, bo0, wo1, bo1, wo2t)


def _stage_c_body(w0_ref, w1_ref, w2_ref, gp0_ref, gp1_ref, gp2_ref, imp_ref):
    g = pl.program_id(0)
    o = _omic(g)
    for i, (w_ref, gp_ref) in enumerate(
            zip((w0_ref, w1_ref, w2_ref), (gp0_ref, gp1_ref, gp2_ref))):
        @pl.when(o == i)
        def _():
            gt = jax.lax.dot_general(w_ref[...], gp_ref[...], _DT,
                                     preferred_element_type=jnp.float32)
            imp_ref[...] = jnp.sum(gt, axis=1, keepdims=True) * (-1.0 / MASK)


def _stage_c(w0, w1, w2, gp0, gp1, gp2):
    gspec = pl.BlockSpec((MASK, H), lambda g: (0, 0))
    return pl.pallas_call(
        _stage_c_body,
        grid=(_C2,),
        in_specs=_WSPECS + [gspec, gspec, gspec],
        out_specs=pl.BlockSpec((BK, 1), lambda g: (g, 0)),
        out_shape=jax.ShapeDtypeStruct((_C2 * BK, 1), jnp.float32),
    )(w0, w1, w2, gp0, gp1, gp2)


def _sortable_key(f):
    b = jax.lax.bitcast_convert_type(f, jnp.uint32)
    neg = b >= jnp.uint32(0x80000000)
    return jnp.where(neg, ~b, b | jnp.uint32(0x80000000))


def _kth_key(key, k):
    kk = jnp.int32(k)

    def bit_step(i, t):
        cand = t | (jnp.uint32(1) << (jnp.uint32(31) - i.astype(jnp.uint32)))
        cnt = jnp.sum((key >= cand).astype(jnp.int32))
        return jnp.where(cnt >= kk, cand, t)

    return jax.lax.fori_loop(0, 32, bit_step, jnp.uint32(0))


def _stage_d_body(imp0_ref, imp1_ref, imp2_ref, mask_ref):
    rows = []
    for ref, fd, k in zip((imp0_ref, imp1_ref, imp2_ref),
                          FEATURE_DIMS, UNMASKED):
        key = _sortable_key(ref[...])
        row = (key >= _kth_key(key, k)).astype(jnp.float32)
        if fd < GRID:
            row = jnp.concatenate(
                [row, jnp.zeros((1, GRID - fd), jnp.float32)], axis=1)
        rows.append(row)
    mask_ref[...] = jnp.concatenate(rows, axis=0)


def _stage_d(imp0, imp1, imp2):
    full = lambda s: pl.BlockSpec(s, lambda: (0, 0))
    return pl.pallas_call(
        _stage_d_body,
        in_specs=[full((1, FEATURE_DIMS[0])), full((1, FEATURE_DIMS[1])),
                  full((1, FEATURE_DIMS[2]))],
        out_specs=full((3, GRID)),
        out_shape=jax.ShapeDtypeStruct((3, GRID), jnp.float32),
    )(imp0, imp1, imp2)


def kernel(x, W0, b0, W1, b1, W2, b2, Wo0, bo0, Wo1, bo1, Wo2, bo2):
    xt = jnp.transpose(x, (0, 2, 1))
    pre0, pre1, pre2 = _stage_a(xt, W0, W1, W2)

    scores, gp0, gp1, gp2 = _stage_b(
        pre0, pre1, pre2, b0[None, :], b1[None, :], b2[None, :],
        Wo0, bo0[None, :], Wo1, bo1[None, :], Wo2.T)

    imp = _stage_c(W0, W1, W2, gp0, gp1, gp2)
    imp_row = imp.reshape(1, _C2 * BK)
    imp0 = jax.lax.slice(imp_row, (0, 0), (1, FEATURE_DIMS[0]))
    imp1 = jax.lax.slice(imp_row, (0, _C0 * BK), (1, _C1 * BK))
    imp2 = jax.lax.slice(imp_row, (0, _C1 * BK), (1, _C2 * BK))
    mask_opt = _stage_d(imp0, imp1, imp2)
    return scores + bo2[None, :], mask_opt


# speedup vs baseline: 2.3343x; 1.0712x over previous
import jax
import jax.numpy as jnp
from jax.experimental import pallas as pl

FEATURE_DIMS = (20000, 5000, 3000)
UNMASKED = (500, 200, 100)
GRID = 20000
MASK = 512
H = 64
BK = 1000
_C0 = FEATURE_DIMS[0] // BK
_C1 = _C0 + FEATURE_DIMS[1] // BK
_C2 = _C1 + FEATURE_DIMS[2] // BK
_DN = (((1,), (0,)), ((), ()))
_DT = (((1,), (1,)), ((), ()))
_D00 = (((0,), (0,)), ((), ()))


def _omic(g):
    return (g >= _C0).astype(jnp.int32) + (g >= _C1).astype(jnp.int32)


def _kloc(g):
    return g - jnp.where(g >= _C1, _C1, jnp.where(g >= _C0, _C0, 0))


_WSPECS = [
    pl.BlockSpec((BK, H), lambda g: (jnp.clip(g, 0, _C0 - 1), 0)),
    pl.BlockSpec((BK, H), lambda g: (jnp.clip(g - _C0, 0, _C1 - _C0 - 1), 0)),
    pl.BlockSpec((BK, H), lambda g: (jnp.clip(g - _C1, 0, _C2 - _C1 - 1), 0)),
]


def _stage_a_body(xt_ref, w0_ref, w1_ref, w2_ref, o0_ref, o1_ref, o2_ref):
    g = pl.program_id(0)
    o = _omic(g)
    xb = xt_ref[...][0]
    for i, (w_ref, out_ref) in enumerate(
            zip((w0_ref, w1_ref, w2_ref), (o0_ref, o1_ref, o2_ref))):
        @pl.when(o == i)
        def _():
            acc = jax.lax.dot_general(
                xb, w_ref[...], _D00, preferred_element_type=jnp.float32)

            @pl.when(_kloc(g) == 0)
            def _():
                out_ref[...] = acc

            @pl.when(_kloc(g) != 0)
            def _():
                out_ref[...] += acc


def _stage_a(xt, w0, w1, w2):
    ospec = pl.BlockSpec((MASK, H), lambda g: (0, 0))
    oshape = jax.ShapeDtypeStruct((MASK, H), jnp.float32)
    return pl.pallas_call(
        _stage_a_body,
        grid=(_C2,),
        in_specs=[pl.BlockSpec((1, BK, MASK),
                               lambda g: (_omic(g), _kloc(g), 0))] + _WSPECS,
        out_specs=[ospec, ospec, ospec],
        out_shape=[oshape, oshape, oshape],
    )(xt, w0, w1, w2)


def _stage_b_body(pre0_ref, pre1_ref, pre2_ref, b0_ref, b1_ref, b2_ref,
                  wo0_ref, bo0_ref, wo1_ref, bo1_ref, wo2t_ref,
                  scores_ref, gp0_ref, gp1_ref, gp2_ref):
    pres = (pre0_ref[...] + b0_ref[...], pre1_ref[...] + b1_ref[...],
            pre2_ref[...] + b2_ref[...])
    ms = tuple(p > 0.0 for p in pres)
    hcat = jnp.concatenate([jnp.maximum(p, 0.0) for p in pres], axis=1)

    a0 = jax.lax.dot_general(hcat, wo0_ref[...], _DN,
                             preferred_element_type=jnp.float32) + bo0_ref[...]
    m0 = a0 > 0.0
    h0 = jnp.maximum(a0, 0.0)

    a1 = jax.lax.dot_general(h0, wo1_ref[...], _DN,
                             preferred_element_type=jnp.float32) + bo1_ref[...]
    m1 = a1 > 0.0
    h1 = jnp.maximum(a1, 0.0)

    wo2t = wo2t_ref[...]
    scores_ref[...] = jax.lax.dot_general(
        h1, wo2t, _DT, preferred_element_type=jnp.float32)

    g1 = jnp.where(m1, wo2t, 0.0)
    g0 = jax.lax.dot_general(g1, wo1_ref[...], _DT,
                             preferred_element_type=jnp.float32)
    g0 = jnp.where(m0, g0, 0.0)
    for i, gp_ref in enumerate((gp0_ref, gp1_ref, gp2_ref)):
        gc_i = jax.lax.dot_general(
            g0, wo0_ref[i * H:(i + 1) * H, :], _DT,
            preferred_element_type=jnp.float32)
        gp_ref[...] = jnp.where(ms[i], gc_i, 0.0)


def _stage_b(pre0, pre1, pre2, b0, b1, b2, wo0, bo0, wo1, bo1, wo2t):
    full = lambda s: pl.BlockSpec(s, lambda: tuple(0 for _ in s))
    gshape = jax.ShapeDtypeStruct((MASK, H), jnp.float32)
    return pl.pallas_call(
        _stage_b_body,
        in_specs=[full((MASK, H)), full((MASK, H)), full((MASK, H)),
                  full((1, H)), full((1, H)), full((1, H)),
                  full((3 * H, 128)), full((1, 128)),
                  full((128, 32)), full((1, 32)), full((1, 32))],
        out_specs=[full((MASK, 1)), full((MASK, H)), full((MASK, H)),
                   full((MASK, H))],
        out_shape=[jax.ShapeDtypeStruct((MASK, 1), jnp.float32),
                   gshape, gshape, gshape],
    )(pre0, pre1, pre2, b0, b1, b2, wo0, bo0, wo1, bo1, wo2t)


def _stage_c_body(w0_ref, w1_ref, w2_ref, gp0_ref, gp1_ref, gp2_ref, imp_ref):
    g = pl.program_id(0)
    o = _omic(g)
    for i, (w_ref, gp_ref) in enumerate(
            zip((w0_ref, w1_ref, w2_ref), (gp0_ref, gp1_ref, gp2_ref))):
        @pl.when(o == i)
        def _():
            gt = jax.lax.dot_general(w_ref[...], gp_ref[...], _DT,
                                     preferred_element_type=jnp.float32)
            imp_ref[...] = jnp.sum(gt, axis=1, keepdims=True) * (-1.0 / MASK)


def _stage_c(w0, w1, w2, gp0, gp1, gp2):
    gspec = pl.BlockSpec((MASK, H), lambda g: (0, 0))
    return pl.pallas_call(
        _stage_c_body,
        grid=(_C2,),
        in_specs=_WSPECS + [gspec, gspec, gspec],
        out_specs=pl.BlockSpec((BK, 1), lambda g: (g, 0)),
        out_shape=jax.ShapeDtypeStruct((_C2 * BK, 1), jnp.float32),
    )(w0, w1, w2, gp0, gp1, gp2)


def _sortable_key(f):
    b = jax.lax.bitcast_convert_type(f, jnp.uint32)
    neg = b >= jnp.uint32(0x80000000)
    return jnp.where(neg, ~b, b | jnp.uint32(0x80000000))


def _kth_keys3(keys):
    ks = tuple(jnp.int32(k) for k in UNMASKED)

    def bit_step(i, ts):
        bit = jnp.uint32(1) << (jnp.uint32(31) - i.astype(jnp.uint32))
        out = []
        for key, t, kk in zip(keys, ts, ks):
            cand = t | bit
            cnt = jnp.sum((key >= cand).astype(jnp.int32))
            out.append(jnp.where(cnt >= kk, cand, t))
        return tuple(out)

    return jax.lax.fori_loop(0, 32, bit_step,
                             (jnp.uint32(0),) * 3)


def _stage_d_body(imp0_ref, imp1_ref, imp2_ref, mask_ref):
    keys = tuple(_sortable_key(ref[...])
                 for ref in (imp0_ref, imp1_ref, imp2_ref))
    kth = _kth_keys3(keys)
    rows = []
    for key, t, fd in zip(keys, kth, FEATURE_DIMS):
        row = (key >= t).astype(jnp.float32)
        if fd < GRID:
            row = jnp.concatenate(
                [row, jnp.zeros((1, GRID - fd), jnp.float32)], axis=1)
        rows.append(row)
    mask_ref[...] = jnp.concatenate(rows, axis=0)


def _stage_d(imp0, imp1, imp2):
    full = lambda s: pl.BlockSpec(s, lambda: (0, 0))
    return pl.pallas_call(
        _stage_d_body,
        in_specs=[full((1, FEATURE_DIMS[0])), full((1, FEATURE_DIMS[1])),
                  full((1, FEATURE_DIMS[2]))],
        out_specs=full((3, GRID)),
        out_shape=jax.ShapeDtypeStruct((3, GRID), jnp.float32),
    )(imp0, imp1, imp2)


def kernel(x, W0, b0, W1, b1, W2, b2, Wo0, bo0, Wo1, bo1, Wo2, bo2):
    xt = jnp.transpose(x, (0, 2, 1))
    pre0, pre1, pre2 = _stage_a(xt, W0, W1, W2)

    scores, gp0, gp1, gp2 = _stage_b(
        pre0, pre1, pre2, b0[None, :], b1[None, :], b2[None, :],
        Wo0, bo0[None, :], Wo1, bo1[None, :], Wo2.T)

    imp = _stage_c(W0, W1, W2, gp0, gp1, gp2)
    imp_row = imp.reshape(1, _C2 * BK)
    imp0 = jax.lax.slice(imp_row, (0, 0), (1, FEATURE_DIMS[0]))
    imp1 = jax.lax.slice(imp_row, (0, _C0 * BK), (1, _C1 * BK))
    imp2 = jax.lax.slice(imp_row, (0, _C1 * BK), (1, _C2 * BK))
    mask_opt = _stage_d(imp0, imp1, imp2)
    return scores + bo2[None, :], mask_opt


# speedup vs baseline: 2.5409x; 1.0885x over previous
import jax
import jax.numpy as jnp
from jax.experimental import pallas as pl

FEATURE_DIMS = (20000, 5000, 3000)
UNMASKED = (500, 200, 100)
GRID = 20000
MASK = 512
H = 64
BK = 1000
_C0 = FEATURE_DIMS[0] // BK
_C1 = _C0 + FEATURE_DIMS[1] // BK
_C2 = _C1 + FEATURE_DIMS[2] // BK
_DN = (((1,), (0,)), ((), ()))
_DT = (((1,), (1,)), ((), ()))
_D00 = (((0,), (0,)), ((), ()))


def _omic(g):
    return (g >= _C0).astype(jnp.int32) + (g >= _C1).astype(jnp.int32)


def _kloc(g):
    return g - jnp.where(g >= _C1, _C1, jnp.where(g >= _C0, _C0, 0))


_WSPECS = [
    pl.BlockSpec((BK, H), lambda g: (jnp.clip(g, 0, _C0 - 1), 0)),
    pl.BlockSpec((BK, H), lambda g: (jnp.clip(g - _C0, 0, _C1 - _C0 - 1), 0)),
    pl.BlockSpec((BK, H), lambda g: (jnp.clip(g - _C1, 0, _C2 - _C1 - 1), 0)),
]


def _stage_a_body(xt_ref, w0_ref, w1_ref, w2_ref, o0_ref, o1_ref, o2_ref):
    g = pl.program_id(0)
    o = _omic(g)
    xb = xt_ref[...][0]
    for i, (w_ref, out_ref) in enumerate(
            zip((w0_ref, w1_ref, w2_ref), (o0_ref, o1_ref, o2_ref))):
        @pl.when(o == i)
        def _():
            acc = jax.lax.dot_general(
                xb, w_ref[...], _D00, preferred_element_type=jnp.float32)

            @pl.when(_kloc(g) == 0)
            def _():
                out_ref[...] = acc

            @pl.when(_kloc(g) != 0)
            def _():
                out_ref[...] += acc


def _stage_a(xt, w0, w1, w2):
    ospec = pl.BlockSpec((MASK, H), lambda g: (0, 0))
    oshape = jax.ShapeDtypeStruct((MASK, H), jnp.float32)
    return pl.pallas_call(
        _stage_a_body,
        grid=(_C2,),
        in_specs=[pl.BlockSpec((1, BK, MASK),
                               lambda g: (_omic(g), _kloc(g), 0))] + _WSPECS,
        out_specs=[ospec, ospec, ospec],
        out_shape=[oshape, oshape, oshape],
    )(xt, w0, w1, w2)


def _stage_b_body(pre0_ref, pre1_ref, pre2_ref, b0_ref, b1_ref, b2_ref,
                  wo0_ref, bo0_ref, wo1_ref, bo1_ref, wo2t_ref,
                  scores_ref, gp0_ref, gp1_ref, gp2_ref):
    pres = (pre0_ref[...] + b0_ref[...], pre1_ref[...] + b1_ref[...],
            pre2_ref[...] + b2_ref[...])
    ms = tuple(p > 0.0 for p in pres)
    hcat = jnp.concatenate([jnp.maximum(p, 0.0) for p in pres], axis=1)

    a0 = jax.lax.dot_general(hcat, wo0_ref[...], _DN,
                             preferred_element_type=jnp.float32) + bo0_ref[...]
    m0 = a0 > 0.0
    h0 = jnp.maximum(a0, 0.0)

    a1 = jax.lax.dot_general(h0, wo1_ref[...], _DN,
                             preferred_element_type=jnp.float32) + bo1_ref[...]
    m1 = a1 > 0.0
    h1 = jnp.maximum(a1, 0.0)

    wo2t = wo2t_ref[...]
    scores_ref[...] = jax.lax.dot_general(
        h1, wo2t, _DT, preferred_element_type=jnp.float32)

    g1 = jnp.where(m1, wo2t, 0.0)
    g0 = jax.lax.dot_general(g1, wo1_ref[...], _DT,
                             preferred_element_type=jnp.float32)
    g0 = jnp.where(m0, g0, 0.0)
    for i, gp_ref in enumerate((gp0_ref, gp1_ref, gp2_ref)):
        gc_i = jax.lax.dot_general(
            g0, wo0_ref[i * H:(i + 1) * H, :], _DT,
            preferred_element_type=jnp.float32)
        gp_ref[...] = jnp.where(ms[i], gc_i, 0.0)


def _stage_b(pre0, pre1, pre2, b0, b1, b2, wo0, bo0, wo1, bo1, wo2t):
    full = lambda s: pl.BlockSpec(s, lambda: tuple(0 for _ in s))
    gshape = jax.ShapeDtypeStruct((MASK, H), jnp.float32)
    return pl.pallas_call(
        _stage_b_body,
        in_specs=[full((MASK, H)), full((MASK, H)), full((MASK, H)),
                  full((1, H)), full((1, H)), full((1, H)),
                  full((3 * H, 128)), full((1, 128)),
                  full((128, 32)), full((1, 32)), full((1, 32))],
        out_specs=[full((MASK, 1)), full((MASK, H)), full((MASK, H)),
                   full((MASK, H))],
        out_shape=[jax.ShapeDtypeStruct((MASK, 1), jnp.float32),
                   gshape, gshape, gshape],
    )(pre0, pre1, pre2, b0, b1, b2, wo0, bo0, wo1, bo1, wo2t)


BKP = 1024


def _stage_c_body(w0_ref, w1_ref, w2_ref, gp0_ref, gp1_ref, gp2_ref, imp_ref):
    g = pl.program_id(0)
    o = _omic(g)
    for i, (w_ref, gp_ref) in enumerate(
            zip((w0_ref, w1_ref, w2_ref), (gp0_ref, gp1_ref, gp2_ref))):
        @pl.when(o == i)
        def _():
            wp = jnp.concatenate(
                [w_ref[...], jnp.zeros((BKP - BK, H), jnp.float32)], axis=0)
            gt = jax.lax.dot_general(gp_ref[...], wp, _DT,
                                     preferred_element_type=jnp.float32)
            imp_ref[...] = (jnp.sum(gt, axis=0, keepdims=True)
                            * (-1.0 / MASK))[None]


def _stage_c(w0, w1, w2, gp0, gp1, gp2):
    gspec = pl.BlockSpec((MASK, H), lambda g: (0, 0))
    return pl.pallas_call(
        _stage_c_body,
        grid=(_C2,),
        in_specs=_WSPECS + [gspec, gspec, gspec],
        out_specs=pl.BlockSpec((1, 1, BKP), lambda g: (g, 0, 0)),
        out_shape=jax.ShapeDtypeStruct((_C2, 1, BKP), jnp.float32),
    )(w0, w1, w2, gp0, gp1, gp2)


def _sortable_key(f):
    b = jax.lax.bitcast_convert_type(f, jnp.uint32)
    neg = b >= jnp.uint32(0x80000000)
    return jnp.where(neg, ~b, b | jnp.uint32(0x80000000))


def _stage_d_body(imp_ref, ind_ref):
    key = _sortable_key(imp_ref[...])
    row = jax.lax.broadcasted_iota(jnp.int32, (_C2, BKP), 0)
    col = jax.lax.broadcasted_iota(jnp.int32, (_C2, BKP), 1)
    valid = col < BK
    o = (row >= _C0).astype(jnp.int32) + (row >= _C1).astype(jnp.int32)
    omasks = tuple((o == i) & valid for i in range(3))
    key = jnp.where(valid, key, jnp.uint32(0))
    ks = tuple(jnp.int32(k) for k in UNMASKED)

    def bit_step(i, ts):
        bit = jnp.uint32(1) << (jnp.uint32(31) - i.astype(jnp.uint32))
        out = []
        for om, t, kk in zip(omasks, ts, ks):
            cand = t | bit
            cnt = jnp.sum(((key >= cand) & om).astype(jnp.int32))
            out.append(jnp.where(cnt >= kk, cand, t))
        return tuple(out)

    t0, t1, t2 = jax.lax.fori_loop(0, 32, bit_step, (jnp.uint32(0),) * 3)
    tsel = jnp.where(o == 0, t0, jnp.where(o == 1, t1, t2))
    ind_ref[...] = jnp.where(valid & (key >= tsel), 1.0, 0.0)


def _stage_d(imp2d):
    full = lambda s: pl.BlockSpec(s, lambda: (0, 0))
    return pl.pallas_call(
        _stage_d_body,
        in_specs=[full((_C2, BKP))],
        out_specs=full((_C2, BKP)),
        out_shape=jax.ShapeDtypeStruct((_C2, BKP), jnp.float32),
    )(imp2d)


def kernel(x, W0, b0, W1, b1, W2, b2, Wo0, bo0, Wo1, bo1, Wo2, bo2):
    xt = jnp.transpose(x, (0, 2, 1))
    pre0, pre1, pre2 = _stage_a(xt, W0, W1, W2)

    scores, gp0, gp1, gp2 = _stage_b(
        pre0, pre1, pre2, b0[None, :], b1[None, :], b2[None, :],
        Wo0, bo0[None, :], Wo1, bo1[None, :], Wo2.T)

    imp = _stage_c(W0, W1, W2, gp0, gp1, gp2)
    ind = _stage_d(imp.reshape(_C2, BKP))
    flat = ind[:, :BK].reshape(_C2 * BK)
    rows = [
        jax.lax.slice(flat, (0,), (FEATURE_DIMS[0],)),
        jnp.pad(jax.lax.slice(flat, (_C0 * BK,), (_C1 * BK,)),
                (0, GRID - FEATURE_DIMS[1])),
        jnp.pad(jax.lax.slice(flat, (_C1 * BK,), (_C2 * BK,)),
                (0, GRID - FEATURE_DIMS[2])),
    ]
    mask_opt = jnp.stack(rows, axis=0)
    return scores + bo2[None, :], mask_opt


# speedup vs baseline: 2.5944x; 1.0210x over previous
import jax
import jax.numpy as jnp
from jax.experimental import pallas as pl

FEATURE_DIMS = (20000, 5000, 3000)
UNMASKED = (500, 200, 100)
GRID = 20000
MASK = 512
H = 64
BK = 1000
_C0 = FEATURE_DIMS[0] // BK
_C1 = _C0 + FEATURE_DIMS[1] // BK
_C2 = _C1 + FEATURE_DIMS[2] // BK
_DN = (((1,), (0,)), ((), ()))
_DT = (((1,), (1,)), ((), ()))
_D00 = (((0,), (0,)), ((), ()))


def _omic(g):
    return (g >= _C0).astype(jnp.int32) + (g >= _C1).astype(jnp.int32)


def _kloc(g):
    return g - jnp.where(g >= _C1, _C1, jnp.where(g >= _C0, _C0, 0))


_WSPECS = [
    pl.BlockSpec((BK, H), lambda g: (jnp.clip(g, 0, _C0 - 1), 0)),
    pl.BlockSpec((BK, H), lambda g: (jnp.clip(g - _C0, 0, _C1 - _C0 - 1), 0)),
    pl.BlockSpec((BK, H), lambda g: (jnp.clip(g - _C1, 0, _C2 - _C1 - 1), 0)),
]


def _stage_a_body(xt_ref, w0_ref, w1_ref, w2_ref, o0_ref, o1_ref, o2_ref):
    g = pl.program_id(0)
    o = _omic(g)
    xb = xt_ref[...][0]
    for i, (w_ref, out_ref) in enumerate(
            zip((w0_ref, w1_ref, w2_ref), (o0_ref, o1_ref, o2_ref))):
        @pl.when(o == i)
        def _():
            acc = jax.lax.dot_general(
                xb, w_ref[...], _D00, preferred_element_type=jnp.float32)

            @pl.when(_kloc(g) == 0)
            def _():
                out_ref[...] = acc

            @pl.when(_kloc(g) != 0)
            def _():
                out_ref[...] += acc


def _stage_a(xt, w0, w1, w2):
    ospec = pl.BlockSpec((MASK, H), lambda g: (0, 0))
    oshape = jax.ShapeDtypeStruct((MASK, H), jnp.float32)
    return pl.pallas_call(
        _stage_a_body,
        grid=(_C2,),
        in_specs=[pl.BlockSpec((1, BK, MASK),
                               lambda g: (_omic(g), _kloc(g), 0))] + _WSPECS,
        out_specs=[ospec, ospec, ospec],
        out_shape=[oshape, oshape, oshape],
    )(xt, w0, w1, w2)


def _stage_b_body(pre0_ref, pre1_ref, pre2_ref, b0_ref, b1_ref, b2_ref,
                  wo0_ref, bo0_ref, wo1_ref, bo1_ref, wo2t_ref,
                  scores_ref, gp0_ref, gp1_ref, gp2_ref):
    pres = (pre0_ref[...] + b0_ref[...], pre1_ref[...] + b1_ref[...],
            pre2_ref[...] + b2_ref[...])
    ms = tuple(p > 0.0 for p in pres)
    hcat = jnp.concatenate([jnp.maximum(p, 0.0) for p in pres], axis=1)

    a0 = jax.lax.dot_general(hcat, wo0_ref[...], _DN,
                             preferred_element_type=jnp.float32) + bo0_ref[...]
    m0 = a0 > 0.0
    h0 = jnp.maximum(a0, 0.0)

    a1 = jax.lax.dot_general(h0, wo1_ref[...], _DN,
                             preferred_element_type=jnp.float32) + bo1_ref[...]
    m1 = a1 > 0.0
    h1 = jnp.maximum(a1, 0.0)

    wo2t = wo2t_ref[...]
    scores_ref[...] = jax.lax.dot_general(
        h1, wo2t, _DT, preferred_element_type=jnp.float32)

    g1 = jnp.where(m1, wo2t, 0.0)
    g0 = jax.lax.dot_general(g1, wo1_ref[...], _DT,
                             preferred_element_type=jnp.float32)
    g0 = jnp.where(m0, g0, 0.0)
    for i, gp_ref in enumerate((gp0_ref, gp1_ref, gp2_ref)):
        gc_i = jax.lax.dot_general(
            g0, wo0_ref[i * H:(i + 1) * H, :], _DT,
            preferred_element_type=jnp.float32)
        gp_ref[...] = jnp.where(ms[i], gc_i, 0.0)


def _stage_b(pre0, pre1, pre2, b0, b1, b2, wo0, bo0, wo1, bo1, wo2t):
    full = lambda s: pl.BlockSpec(s, lambda: tuple(0 for _ in s))
    gshape = jax.ShapeDtypeStruct((MASK, H), jnp.float32)
    return pl.pallas_call(
        _stage_b_body,
        in_specs=[full((MASK, H)), full((MASK, H)), full((MASK, H)),
                  full((1, H)), full((1, H)), full((1, H)),
                  full((3 * H, 128)), full((1, 128)),
                  full((128, 32)), full((1, 32)), full((1, 32))],
        out_specs=[full((MASK, 1)), full((MASK, H)), full((MASK, H)),
                   full((MASK, H))],
        out_shape=[jax.ShapeDtypeStruct((MASK, 1), jnp.float32),
                   gshape, gshape, gshape],
    )(pre0, pre1, pre2, b0, b1, b2, wo0, bo0, wo1, bo1, wo2t)


_CTILES = ((5, 4000, 4096), (1, 5000, 5120), (1, 3000, 3072))


def _stage_c_omic_body(w_ref, gp_ref, imp_ref, *, bk, bkp):
    wp = jnp.concatenate(
        [w_ref[...], jnp.zeros((bkp - bk, H), jnp.float32)], axis=0)
    gt = jax.lax.dot_general(gp_ref[...], wp, _DT,
                             preferred_element_type=jnp.float32)
    imp_ref[...] = (jnp.sum(gt, axis=0, keepdims=True) * (-1.0 / MASK))[None]


def _stage_c_omic(w, gp, nt, bk, bkp):
    import functools
    return pl.pallas_call(
        functools.partial(_stage_c_omic_body, bk=bk, bkp=bkp),
        grid=(nt,),
        in_specs=[pl.BlockSpec((bk, H), lambda g: (g, 0)),
                  pl.BlockSpec((MASK, H), lambda g: (0, 0))],
        out_specs=pl.BlockSpec((1, 1, bkp), lambda g: (g, 0, 0)),
        out_shape=jax.ShapeDtypeStruct((nt, 1, bkp), jnp.float32),
    )(w, gp)


def _sortable_key(f):
    b = jax.lax.bitcast_convert_type(f, jnp.uint32)
    neg = b >= jnp.uint32(0x80000000)
    return jnp.where(neg, ~b, b | jnp.uint32(0x80000000))


def _stage_d_body(imp0_ref, imp1_ref, imp2_ref, i0_ref, i1_ref, i2_ref):
    refs = (imp0_ref, imp1_ref, imp2_ref)
    keys, valids = [], []
    for ref, (nt, bk, bkp) in zip(refs, _CTILES):
        col = jax.lax.broadcasted_iota(jnp.int32, (nt, bkp), 1)
        valid = col < bk
        keys.append(jnp.where(valid, _sortable_key(ref[...]), jnp.uint32(0)))
        valids.append(valid)
    ks = tuple(jnp.int32(k) for k in UNMASKED)

    def bit_step(i, ts):
        bit = jnp.uint32(1) << (jnp.uint32(31) - i.astype(jnp.uint32))
        out = []
        for key, t, kk in zip(keys, ts, ks):
            cand = t | bit
            cnt = jnp.sum((key >= cand).astype(jnp.int32))
            out.append(jnp.where(cnt >= kk, cand, t))
        return tuple(out)

    ts = jax.lax.fori_loop(0, 32, bit_step, (jnp.uint32(0),) * 3)
    for key, valid, t, out_ref in zip(keys, valids, ts,
                                      (i0_ref, i1_ref, i2_ref)):
        out_ref[...] = jnp.where(valid & (key >= t), 1.0, 0.0)


def _stage_d(imp0, imp1, imp2):
    full = lambda s: pl.BlockSpec(s, lambda: (0, 0))
    specs = [full((nt, bkp)) for nt, bk, bkp in _CTILES]
    return pl.pallas_call(
        _stage_d_body,
        in_specs=specs,
        out_specs=specs,
        out_shape=[jax.ShapeDtypeStruct((nt, bkp), jnp.float32)
                   for nt, bk, bkp in _CTILES],
    )(imp0, imp1, imp2)


def kernel(x, W0, b0, W1, b1, W2, b2, Wo0, bo0, Wo1, bo1, Wo2, bo2):
    xt = jnp.transpose(x, (0, 2, 1))
    pre0, pre1, pre2 = _stage_a(xt, W0, W1, W2)

    scores, gp0, gp1, gp2 = _stage_b(
        pre0, pre1, pre2, b0[None, :], b1[None, :], b2[None, :],
        Wo0, bo0[None, :], Wo1, bo1[None, :], Wo2.T)

    imps = [
        _stage_c_omic(w, gp, nt, bk, bkp).reshape(nt, bkp)
        for (w, gp), (nt, bk, bkp) in zip(
            ((W0, gp0), (W1, gp1), (W2, gp2)), _CTILES)
    ]
    inds = _stage_d(*imps)
    rows = []
    for ind, fd, (nt, bk, bkp) in zip(inds, FEATURE_DIMS, _CTILES):
        row = ind[:, :bk].reshape(fd)
        if fd < GRID:
            row = jnp.pad(row, (0, GRID - fd))
        rows.append(row)
    mask_opt = jnp.stack(rows, axis=0)
    return scores + bo2[None, :], mask_opt
